# Initial kernel scaffold; baseline (speedup 1.0000x reference)
import jax
import jax.numpy as jnp
from jax.experimental import pallas as pl


def kernel(x, edge_index, down_W0, down_b0, down_W1, down_b1, down_W2, down_b2, down_W3, down_b3, up_W0, up_b0, up_W1, up_b1, up_W2, up_b2, pool_w0, pool_w1, pool_w2):
    raise NotImplementedError("write your pallas kernel here")


# speedup vs baseline: 1.9345x; 1.9345x over previous
import functools
import math

import jax
import jax.numpy as jnp
from jax.experimental import pallas as pl
from jax.experimental.pallas import tpu as pltpu

_N = 10000
_NP = 10240
_F = 128


def _mm_kernel(a_ref, b_ref, o_ref, acc_ref, *, nk, zero_diag, nt):
    kk = pl.program_id(2)

    @pl.when(kk == 0)
    def _init():
        acc_ref[...] = jnp.zeros_like(acc_ref)

    a = a_ref[...]
    b = b_ref[...]
    if nt:
        acc_ref[...] += jax.lax.dot_general(
            a, b, (((1,), (1,)), ((), ())), preferred_element_type=jnp.float32)
    else:
        acc_ref[...] += jnp.dot(a, b, preferred_element_type=jnp.float32)

    @pl.when(kk == nk - 1)
    def _done():
        out = acc_ref[...]
        if zero_diag:
            i = pl.program_id(0)
            j = pl.program_id(1)
            rr = jax.lax.broadcasted_iota(jnp.int32, out.shape, 0)
            cc = jax.lax.broadcasted_iota(jnp.int32, out.shape, 1)
            out = jnp.where(jnp.logical_and(i == j, rr == cc), 0.0, out)
        o_ref[...] = out


def _mm(a, b, *, nt=False, zero_diag=False):
    m, k = a.shape
    n = b.shape[0] if nt else b.shape[1]
    bm = 1280 if m % 1280 == 0 else m
    bk = 1280 if k % 1280 == 0 else k
    bn = 1280 if n % 1280 == 0 else n
    nk = k // bk
    grid = (m // bm, n // bn, nk)
    if nt:
        in_specs = [pl.BlockSpec((bm, bk), lambda i, j, q: (i, q)),
                    pl.BlockSpec((bn, bk), lambda i, j, q: (j, q))]
    else:
        in_specs = [pl.BlockSpec((bm, bk), lambda i, j, q: (i, q)),
                    pl.BlockSpec((bk, bn), lambda i, j, q: (q, j))]
    return pl.pallas_call(
        functools.partial(_mm_kernel, nk=nk, zero_diag=zero_diag, nt=nt),
        grid=grid,
        in_specs=in_specs,
        out_specs=pl.BlockSpec((bm, bn), lambda i, j, q: (i, j)),
        out_shape=jax.ShapeDtypeStruct((m, n), jnp.float32),
        scratch_shapes=[pltpu.VMEM((bm, bn), jnp.float32)],
        compiler_params=pltpu.CompilerParams(
            dimension_semantics=("parallel", "parallel", "arbitrary")),
    )(a, b)


def _dinv(deg):
    return jnp.where(deg > 0.0, 1.0 / jnp.sqrt(jnp.maximum(deg, 1e-12)), 0.0)


def _gcn_pooled(h, A, W, b):
    deg = A.sum(axis=1) + 2.0
    di = _dinv(deg)
    u = di[:, None] * _mm(h, W)
    Av = _mm(A, u)
    return di[:, None] * (Av + 2.0 * u) + b


def _pool_scores(h, pw, n_real):
    s = jnp.tanh(jnp.dot(h, pw) / jnp.linalg.norm(pw))
    return jnp.where(jnp.arange(h.shape[0]) < n_real, s, -2.0)


def _pad_idx(perm, vals, kpad):
    k = perm.shape[0]
    idxp = jnp.zeros((kpad,), jnp.int32).at[:k].set(perm)
    valsp = jnp.zeros((kpad,), jnp.float32).at[:k].set(vals)
    valid = jnp.arange(kpad) < k
    return idxp, valsp, valid


def _pool_adj(A, At, perm, kpad):
    k = perm.shape[0]
    idxp, _, valid = _pad_idx(perm, jnp.zeros((k,), jnp.float32), kpad)
    r = jnp.arange(k)
    G = jnp.where(valid[:, None], A[idxp, :], 0.0)
    G = G.at[r, perm].set(1.0)
    H = jnp.where(valid[:, None], At[idxp, :], 0.0)
    H = H.at[r, perm].set(1.0)
    exact_bf16 = A.shape[0] >= 5120
    if exact_bf16:
        Anew = _mm(G.astype(jnp.bfloat16), H.astype(jnp.bfloat16),
                   nt=True, zero_diag=True)
    else:
        Anew = _mm(G, H, nt=True, zero_diag=True)
    return Anew


def kernel(x, edge_index, down_W0, down_b0, down_W1, down_b1, down_W2,
           down_b2, down_W3, down_b3, up_W0, up_b0, up_W1, up_b1, up_W2,
           up_b2, pool_w0, pool_w1, pool_w2):
    src = edge_index[0]
    dst = edge_index[1]

    A0 = jnp.zeros((_NP, _NP), jnp.float32).at[dst, src].add(1.0)
    A0t = jnp.zeros((_NP, _NP), jnp.float32).at[src, dst].add(1.0)
    deg_in = jnp.zeros((_NP,), jnp.float32).at[dst].add(1.0)
    c0 = jnp.zeros((_NP,), jnp.float32).at[dst].add(
        (src == dst).astype(jnp.float32))
    s0 = jnp.where(c0 == 0.0, 2.0, 0.0)
    deg0 = deg_in + s0
    di0 = _dinv(deg0)

    xp = jnp.zeros((_NP, _F), jnp.float32).at[:_N].set(x)

    u = di0[:, None] * _mm(xp, down_W0)
    h0 = di0[:, None] * (_mm(A0, u) + s0[:, None] * u) + down_b0
    h0 = jax.nn.relu(h0)

    k1, k1p = 5000, 5120
    sc1 = _pool_scores(h0, pool_w0, _N)
    vals1, perm1 = jax.lax.top_k(sc1, k1)
    idx1, v1, valid1 = _pad_idx(perm1, vals1, k1p)
    hp1 = h0[idx1] * v1[:, None]
    A1 = _pool_adj(A0, A0t, perm1, k1p)
    h1 = jax.nn.relu(_gcn_pooled(hp1, A1, down_W1, down_b1))

    k2, k2p = 2500, 2560
    A1t = A1.T
    sc2 = _pool_scores(h1, pool_w1, k1)
    vals2, perm2 = jax.lax.top_k(sc2, k2)
    idx2, v2, valid2 = _pad_idx(perm2, vals2, k2p)
    hp2 = h1[idx2] * v2[:, None]
    A2 = _pool_adj(A1, A1t, perm2, k2p)
    h2 = jax.nn.relu(_gcn_pooled(hp2, A2, down_W2, down_b2))

    k3, k3p = 1250, 1280
    A2t = A2.T
    sc3 = _pool_scores(h2, pool_w2, k2)
    vals3, perm3 = jax.lax.top_k(sc3, k3)
    idx3, v3, valid3 = _pad_idx(perm3, vals3, k3p)
    hp3 = h2[idx3] * v3[:, None]
    A3 = _pool_adj(A2, A2t, perm3, k3p)
    h3 = jax.nn.relu(_gcn_pooled(hp3, A3, down_W3, down_b3))

    u0 = h2 + jnp.zeros_like(h2).at[perm3].set(h3[:k3])
    g = jax.nn.relu(_gcn_pooled(u0, A2, up_W0, up_b0))

    u1 = h1 + jnp.zeros_like(h1).at[perm2].set(g[:k2])
    g = jax.nn.relu(_gcn_pooled(u1, A1, up_W1, up_b1))

    u2 = h0 + jnp.zeros_like(h0).at[perm1].set(g[:k1])
    v = di0[:, None] * _mm(u2, up_W2)
    out = di0[:, None] * (_mm(A0, v) + s0[:, None] * v) + up_b2

    return out[:_N]
